# baseline (speedup 1.0000x reference)
import functools

import jax
import jax.numpy as jnp
from jax import lax
from jax.experimental import pallas as pl
from jax.experimental.pallas import tpu as pltpu
from jax.experimental.pallas import tpu_sc as plsc


def _table_body(emb_ref, w_ref, b_ref, g_ref, beta_ref, out_ref):
    h = lax.dot_general(
        emb_ref[...], w_ref[...], (((1,), (1,)), ((), ())),
        preferred_element_type=jnp.float32,
    )
    h = h + b_ref[...]
    mu = jnp.mean(h, axis=-1, keepdims=True)
    d = h - mu
    var = jnp.mean(d * d, axis=-1, keepdims=True)
    t = (d * lax.rsqrt(var + 1e-5)) * g_ref[...] + beta_ref[...]
    V, D = t.shape
    left = jnp.broadcast_to(t[:, None, :], (V, V, D))
    right = jnp.broadcast_to(t[None, :, :], (V, V, D))
    out_ref[...] = jnp.concatenate([left, right], axis=-1)


def _make_paired_table(emb, W, b, gamma, beta):
    V, D = emb.shape
    paired = pl.pallas_call(
        _table_body,
        out_shape=jax.ShapeDtypeStruct((V, V, 2 * D), jnp.float32),
    )(emb, W, b.reshape(1, D), gamma.reshape(1, D), beta.reshape(1, D))
    return paired.reshape(V * V, 2 * D)


_NSLOT = 2


def _make_gather(N2, VV, n_workers, chunk):
    n_per_w = N2 // n_workers
    n_chunks = n_per_w // chunk
    n_outer = n_chunks // _NSLOT
    n_groups = chunk // 16
    mesh = plsc.VectorSubcoreMesh(core_axis_name="c", subcore_axis_name="s")

    scratch = (
        [pltpu.VMEM((VV * 128,), jnp.float32),
         pltpu.VMEM((n_per_w,), jnp.int32)]
        + [pltpu.VMEM((chunk * 128,), jnp.float32) for _ in range(_NSLOT)]
        + [pltpu.SemaphoreType.DMA for _ in range(_NSLOT + 1)]
    )

    @functools.partial(
        pl.kernel,
        out_type=jax.ShapeDtypeStruct((N2 * 128,), jnp.float32),
        mesh=mesh,
        scratch_types=scratch,
        compiler_params=pltpu.CompilerParams(needs_layout_passes=False),
    )
    def gather_k(tab_hbm, idx_hbm, out_hbm, *refs):
        tab_v = refs[0]
        idx_v = refs[1]
        rows = refs[2:2 + _NSLOT]
        sem_s = refs[2 + _NSLOT:2 + 2 * _NSLOT]
        sem_ld = refs[2 + 2 * _NSLOT]

        wid = lax.axis_index("s") * 2 + lax.axis_index("c")
        base = pl.multiple_of(wid * n_per_w, n_per_w)

        pltpu.async_copy(tab_hbm, tab_v, sem_ld)
        pltpu.async_copy(idx_hbm.at[pl.ds(base, n_per_w)], idx_v, sem_ld)
        pltpu.make_async_copy(tab_hbm, tab_v, sem_ld).wait()
        pltpu.make_async_copy(
            idx_hbm.at[pl.ds(base, n_per_w)], idx_v, sem_ld
        ).wait()

        lane128 = lax.iota(jnp.int32, 16) * 128

        def expand_chunk(g, buf):
            def group(q, carry):
                iv = idx_v[pl.ds(g * chunk + q * 16, 16)]
                ga = iv * 128
                sa = lane128 + q * 2048
                for _c in range(128):
                    vals = plsc.load_gather(tab_v, [ga])
                    plsc.store_scatter(buf, [sa], vals)
                    ga = ga + 1
                    sa = sa + 1
                return carry

            lax.fori_loop(0, n_groups, group, 0)

        def fire_store(g, b):
            pltpu.async_copy(
                rows[b],
                out_hbm.at[pl.ds((base + g * chunk) * 128, chunk * 128)],
                sem_s[b],
            )

        def wait_store(b):
            pltpu.make_async_copy(
                rows[b], out_hbm.at[pl.ds(base * 128, chunk * 128)], sem_s[b]
            ).wait()

        def outer(i, carry):
            g0 = i * _NSLOT
            for b in range(_NSLOT):
                g = g0 + b

                @pl.when(i > 0)
                def _():
                    wait_store(b)

                expand_chunk(g, rows[b])
                fire_store(g, b)
            return carry

        lax.fori_loop(0, n_outer, outer, 0)
        for b in range(_NSLOT):
            wait_store(b)

    return gather_k


def kernel(x, emb, W, b, gamma, beta):
    B, L = x.shape
    V, D = emb.shape
    N2 = (B * L) // 2
    paired = _make_paired_table(emb, W, b, gamma, beta).reshape(-1)
    xf = x.astype(jnp.int32).reshape(N2, 2)
    idx2 = xf[:, 0] * V + xf[:, 1]
    gather = _make_gather(N2, V * V, n_workers=32, chunk=256)
    out2 = gather(paired, idx2)
    return out2.reshape(B, L, D)


# speedup vs baseline: 1.2066x; 1.2066x over previous
import functools

import jax
import jax.numpy as jnp
from jax import lax
from jax.experimental import pallas as pl
from jax.experimental.pallas import tpu as pltpu
from jax.experimental.pallas import tpu_sc as plsc


def _table_body(emb_ref, w_ref, b_ref, g_ref, beta_ref, out_ref):
    h = lax.dot_general(
        emb_ref[...], w_ref[...], (((1,), (1,)), ((), ())),
        preferred_element_type=jnp.float32,
    )
    h = h + b_ref[...]
    mu = jnp.mean(h, axis=-1, keepdims=True)
    d = h - mu
    var = jnp.mean(d * d, axis=-1, keepdims=True)
    t = (d * lax.rsqrt(var + 1e-5)) * g_ref[...] + beta_ref[...]
    V, D = t.shape
    left = jnp.broadcast_to(t[:, None, :], (V, V, D))
    right = jnp.broadcast_to(t[None, :, :], (V, V, D))
    out_ref[...] = jnp.concatenate([left, right], axis=-1)


def _make_paired_table(emb, W, b, gamma, beta):
    V, D = emb.shape
    paired = pl.pallas_call(
        _table_body,
        out_shape=jax.ShapeDtypeStruct((V, V, 2 * D), jnp.float32),
    )(emb, W, b.reshape(1, D), gamma.reshape(1, D), beta.reshape(1, D))
    return paired.reshape(V * V, 2 * D)


_NSLOT = 2


def _make_gather(N2, VV, n_workers, chunk):
    n_per_w = N2 // n_workers
    n_chunks = n_per_w // chunk
    n_outer = n_chunks // _NSLOT
    n_groups = chunk // 16
    mesh = plsc.VectorSubcoreMesh(core_axis_name="c", subcore_axis_name="s")

    scratch = (
        [pltpu.VMEM((VV * 128,), jnp.float32),
         pltpu.VMEM((n_per_w,), jnp.int32)]
        + [pltpu.VMEM((chunk * 128,), jnp.float32) for _ in range(_NSLOT)]
        + [pltpu.SemaphoreType.DMA for _ in range(_NSLOT + 1)]
    )

    @functools.partial(
        pl.kernel,
        out_type=jax.ShapeDtypeStruct((N2 * 128,), jnp.float32),
        mesh=mesh,
        scratch_types=scratch,
        compiler_params=pltpu.CompilerParams(needs_layout_passes=False),
    )
    def gather_k(tab_hbm, idx_hbm, out_hbm, *refs):
        tab_v = refs[0]
        idx_v = refs[1]
        rows = refs[2:2 + _NSLOT]
        sem_s = refs[2 + _NSLOT:2 + 2 * _NSLOT]
        sem_ld = refs[2 + 2 * _NSLOT]

        wid = lax.axis_index("s") * 2 + lax.axis_index("c")
        base = pl.multiple_of(wid * n_per_w, n_per_w)

        pltpu.async_copy(tab_hbm, tab_v, sem_ld)
        pltpu.async_copy(idx_hbm.at[pl.ds(base, n_per_w)], idx_v, sem_ld)
        pltpu.make_async_copy(tab_hbm, tab_v, sem_ld).wait()
        pltpu.make_async_copy(
            idx_hbm.at[pl.ds(base, n_per_w)], idx_v, sem_ld
        ).wait()

        lane128 = lax.iota(jnp.int32, 16) * 128

        def expand_chunk(g, buf):
            @plsc.parallel_loop(0, n_groups, 1, unroll=2)
            def group(q):
                iv = idx_v[pl.ds(g * chunk + q * 16, 16)]
                ga = iv * 128
                sa = lane128 + q * 2048
                for _c in range(128):
                    vals = plsc.load_gather(tab_v, [ga])
                    plsc.store_scatter(buf, [sa], vals)
                    ga = ga + 1
                    sa = sa + 1

        def fire_store(g, b):
            pltpu.async_copy(
                rows[b],
                out_hbm.at[pl.ds((base + g * chunk) * 128, chunk * 128)],
                sem_s[b],
            )

        def wait_store(b):
            pltpu.make_async_copy(
                rows[b], out_hbm.at[pl.ds(base * 128, chunk * 128)], sem_s[b]
            ).wait()

        def outer(i, carry):
            g0 = i * _NSLOT
            for b in range(_NSLOT):
                g = g0 + b

                @pl.when(i > 0)
                def _():
                    wait_store(b)

                expand_chunk(g, rows[b])
                fire_store(g, b)
            return carry

        lax.fori_loop(0, n_outer, outer, 0)
        for b in range(_NSLOT):
            wait_store(b)

    return gather_k


def kernel(x, emb, W, b, gamma, beta):
    B, L = x.shape
    V, D = emb.shape
    N2 = (B * L) // 2
    paired = _make_paired_table(emb, W, b, gamma, beta).reshape(-1)
    xf = x.astype(jnp.int32).reshape(N2, 2)
    idx2 = xf[:, 0] * V + xf[:, 1]
    gather = _make_gather(N2, V * V, n_workers=32, chunk=256)
    out2 = gather(paired, idx2)
    return out2.reshape(B, L, D)


# speedup vs baseline: 2.9721x; 2.4633x over previous
import functools

import jax
import jax.numpy as jnp
from jax import lax
from jax.experimental import pallas as pl
from jax.experimental.pallas import tpu as pltpu
from jax.experimental.pallas import tpu_sc as plsc


def _table_body(emb_ref, w_ref, b_ref, g_ref, beta_ref, out_ref):
    h = lax.dot_general(
        emb_ref[...], w_ref[...], (((1,), (1,)), ((), ())),
        preferred_element_type=jnp.float32,
    )
    h = h + b_ref[...]
    mu = jnp.mean(h, axis=-1, keepdims=True)
    d = h - mu
    var = jnp.mean(d * d, axis=-1, keepdims=True)
    t = (d * lax.rsqrt(var + 1e-5)) * g_ref[...] + beta_ref[...]
    V, D = t.shape
    left = jnp.broadcast_to(t[:, None, :], (V, V, D))
    right = jnp.broadcast_to(t[None, :, :], (V, V, D))
    out_ref[...] = jnp.concatenate([left, right], axis=-1)


def _make_paired_table(emb, W, b, gamma, beta):
    V, D = emb.shape
    paired = pl.pallas_call(
        _table_body,
        out_shape=jax.ShapeDtypeStruct((V, V, 2 * D), jnp.float32),
    )(emb, W, b.reshape(1, D), gamma.reshape(1, D), beta.reshape(1, D))
    return paired.reshape(V * V, 2 * D)


_NSLOT = 2


def _make_gather(N2, VV, n_workers, chunk):
    n_per_w = N2 // n_workers
    n_chunks = n_per_w // chunk
    n_outer = n_chunks // _NSLOT
    n_groups = chunk // 16
    mesh = plsc.VectorSubcoreMesh(core_axis_name="c", subcore_axis_name="s")

    scratch = (
        [pltpu.VMEM((VV * 128,), jnp.float32),
         pltpu.VMEM((n_per_w,), jnp.int32)]
        + [pltpu.VMEM((chunk * 128,), jnp.float32) for _ in range(_NSLOT)]
        + [pltpu.SemaphoreType.DMA for _ in range(_NSLOT + 1)]
    )

    @functools.partial(
        pl.kernel,
        out_type=jax.ShapeDtypeStruct((N2 * 128,), jnp.float32),
        mesh=mesh,
        scratch_types=scratch,
        compiler_params=pltpu.CompilerParams(needs_layout_passes=False),
    )
    def gather_k(tab_hbm, idx_hbm, out_hbm, *refs):
        tab_v = refs[0]
        idx_v = refs[1]
        rows = refs[2:2 + _NSLOT]
        sem_s = refs[2 + _NSLOT:2 + 2 * _NSLOT]
        sem_ld = refs[2 + 2 * _NSLOT]

        wid = lax.axis_index("s") * 2 + lax.axis_index("c")
        base = pl.multiple_of(wid * n_per_w, n_per_w)

        pltpu.async_copy(tab_hbm, tab_v, sem_ld)
        pltpu.async_copy(idx_hbm.at[pl.ds(base, n_per_w)], idx_v, sem_ld)
        pltpu.make_async_copy(tab_hbm, tab_v, sem_ld).wait()
        pltpu.make_async_copy(
            idx_hbm.at[pl.ds(base, n_per_w)], idx_v, sem_ld
        ).wait()

        def expand_chunk(g, buf):
            @plsc.parallel_loop(0, n_groups, 1, unroll=2)
            def group(q):
                iv = idx_v[pl.ds(g * chunk + q * 16, 16)]
                ob0 = q * 2048
                for j in range(16):
                    tb = iv[j] * 128
                    ob = ob0 + j * 128
                    for k in range(8):
                        buf[pl.ds(ob + k * 16, 16)] = (
                            tab_v[pl.ds(tb + k * 16, 16)]
                        )

        def fire_store(g, b):
            pltpu.async_copy(
                rows[b],
                out_hbm.at[pl.ds((base + g * chunk) * 128, chunk * 128)],
                sem_s[b],
            )

        def wait_store(b):
            pltpu.make_async_copy(
                rows[b], out_hbm.at[pl.ds(base * 128, chunk * 128)], sem_s[b]
            ).wait()

        def outer(i, carry):
            g0 = i * _NSLOT
            for b in range(_NSLOT):
                g = g0 + b

                @pl.when(i > 0)
                def _():
                    wait_store(b)

                expand_chunk(g, rows[b])
                fire_store(g, b)
            return carry

        lax.fori_loop(0, n_outer, outer, 0)
        for b in range(_NSLOT):
            wait_store(b)

    return gather_k


def kernel(x, emb, W, b, gamma, beta):
    B, L = x.shape
    V, D = emb.shape
    N2 = (B * L) // 2
    paired = _make_paired_table(emb, W, b, gamma, beta).reshape(-1)
    xf = x.astype(jnp.int32).reshape(N2, 2)
    idx2 = xf[:, 0] * V + xf[:, 1]
    gather = _make_gather(N2, V * V, n_workers=32, chunk=256)
    out2 = gather(paired, idx2)
    return out2.reshape(B, L, D)


# speedup vs baseline: 3.2659x; 1.0989x over previous
import functools

import jax
import jax.numpy as jnp
from jax import lax
from jax.experimental import pallas as pl
from jax.experimental.pallas import tpu as pltpu
from jax.experimental.pallas import tpu_sc as plsc


def _table_body(emb_ref, w_ref, b_ref, g_ref, beta_ref, out_ref):
    h = lax.dot_general(
        emb_ref[...], w_ref[...], (((1,), (1,)), ((), ())),
        preferred_element_type=jnp.float32,
    )
    h = h + b_ref[...]
    mu = jnp.mean(h, axis=-1, keepdims=True)
    d = h - mu
    var = jnp.mean(d * d, axis=-1, keepdims=True)
    t = (d * lax.rsqrt(var + 1e-5)) * g_ref[...] + beta_ref[...]
    V, D = t.shape
    left = jnp.broadcast_to(t[:, None, :], (V, V, D))
    right = jnp.broadcast_to(t[None, :, :], (V, V, D))
    out_ref[...] = jnp.concatenate([left, right], axis=-1)


def _make_paired_table(emb, W, b, gamma, beta):
    V, D = emb.shape
    paired = pl.pallas_call(
        _table_body,
        out_shape=jax.ShapeDtypeStruct((V, V, 2 * D), jnp.float32),
    )(emb, W, b.reshape(1, D), gamma.reshape(1, D), beta.reshape(1, D))
    return paired.reshape(V * V, 2 * D)


_NSLOT = 2


def _make_gather(N2, VV, n_workers, chunk):
    n_per_w = N2 // n_workers
    n_chunks = n_per_w // chunk
    n_outer = n_chunks // _NSLOT
    n_groups = chunk // 16
    mesh = plsc.VectorSubcoreMesh(core_axis_name="c", subcore_axis_name="s")

    n_per_sc = n_per_w * 16
    scratch = (
        [pltpu.VMEM((VV * 128,), jnp.float32),
         pltpu.VMEM_SHARED((n_per_sc,), jnp.int32),
         pltpu.SMEM((chunk,), jnp.int32)]
        + [pltpu.VMEM((chunk * 128,), jnp.float32) for _ in range(_NSLOT)]
        + [pltpu.SemaphoreType.DMA for _ in range(_NSLOT + 1)]
    )

    @functools.partial(
        pl.kernel,
        out_type=jax.ShapeDtypeStruct((N2 * 128,), jnp.float32),
        mesh=mesh,
        scratch_types=scratch,
        compiler_params=pltpu.CompilerParams(needs_layout_passes=False),
    )
    def gather_k(tab_hbm, idx_hbm, out_hbm, *refs):
        tab_v = refs[0]
        idx_sh = refs[1]
        idx_sm = refs[2]
        rows = refs[3:3 + _NSLOT]
        sem_s = refs[3 + _NSLOT:3 + 2 * _NSLOT]
        sem_ld = refs[3 + 2 * _NSLOT]

        cid = lax.axis_index("c")
        sid = lax.axis_index("s")
        wid = cid * 16 + sid
        base = pl.multiple_of(wid * n_per_w, n_per_w)
        sbase = pl.multiple_of(sid * n_per_w, n_per_w)

        pltpu.async_copy(tab_hbm, tab_v, sem_ld)

        @pl.when(sid == 0)
        def _():
            scb = pl.multiple_of(cid * n_per_sc, n_per_sc)
            pltpu.sync_copy(idx_hbm.at[pl.ds(scb, n_per_sc)], idx_sh)

        pltpu.make_async_copy(tab_hbm, tab_v, sem_ld).wait()
        plsc.subcore_barrier()

        def expand_chunk(g, buf):
            pltpu.sync_copy(
                idx_sh.at[pl.ds(sbase + g * chunk, chunk)], idx_sm
            )

            @plsc.parallel_loop(0, chunk, 1, unroll=2)
            def pair(p):
                tb = idx_sm[p] * 128
                ob = p * 128
                for k in range(8):
                    buf[pl.ds(ob + k * 16, 16)] = tab_v[pl.ds(tb + k * 16, 16)]

        def fire_store(g, b):
            pltpu.async_copy(
                rows[b],
                out_hbm.at[pl.ds((base + g * chunk) * 128, chunk * 128)],
                sem_s[b],
            )

        def wait_store(b):
            pltpu.make_async_copy(
                rows[b], out_hbm.at[pl.ds(base * 128, chunk * 128)], sem_s[b]
            ).wait()

        def outer(i, carry):
            g0 = i * _NSLOT
            for b in range(_NSLOT):
                g = g0 + b

                @pl.when(i > 0)
                def _():
                    wait_store(b)

                expand_chunk(g, rows[b])
                fire_store(g, b)
            return carry

        lax.fori_loop(0, n_outer, outer, 0)
        for b in range(_NSLOT):
            wait_store(b)

    return gather_k


def kernel(x, emb, W, b, gamma, beta):
    B, L = x.shape
    V, D = emb.shape
    N2 = (B * L) // 2
    paired = _make_paired_table(emb, W, b, gamma, beta).reshape(-1)
    xf = x.astype(jnp.int32).reshape(N2, 2)
    idx2 = xf[:, 0] * V + xf[:, 1]
    gather = _make_gather(N2, V * V, n_workers=32, chunk=256)
    out2 = gather(paired, idx2)
    return out2.reshape(B, L, D)


# speedup vs baseline: 4.3825x; 1.3419x over previous
import functools

import jax
import jax.numpy as jnp
from jax import lax
from jax.experimental import pallas as pl
from jax.experimental.pallas import tpu as pltpu
from jax.experimental.pallas import tpu_sc as plsc


def _table_body(emb_ref, w_ref, b_ref, g_ref, beta_ref, out_ref):
    h = lax.dot_general(
        emb_ref[...], w_ref[...], (((1,), (1,)), ((), ())),
        preferred_element_type=jnp.float32,
    )
    h = h + b_ref[...]
    mu = jnp.mean(h, axis=-1, keepdims=True)
    d = h - mu
    var = jnp.mean(d * d, axis=-1, keepdims=True)
    t = (d * lax.rsqrt(var + 1e-5)) * g_ref[...] + beta_ref[...]
    V, D = t.shape
    left = jnp.broadcast_to(t[:, None, :], (V, V, D))
    right = jnp.broadcast_to(t[None, :, :], (V, V, D))
    out_ref[...] = jnp.concatenate([left, right], axis=-1)


def _make_paired_table(emb, W, b, gamma, beta):
    V, D = emb.shape
    paired = pl.pallas_call(
        _table_body,
        out_shape=jax.ShapeDtypeStruct((V, V, 2 * D), jnp.float32),
    )(emb, W, b.reshape(1, D), gamma.reshape(1, D), beta.reshape(1, D))
    return paired.reshape(V * V, 2 * D)


_NSLOT = 2


def _make_gather(N2, VV, n_workers, chunk):
    n_per_w = N2 // n_workers
    n_chunks = n_per_w // chunk
    n_outer = n_chunks // _NSLOT
    n_groups = chunk // 16
    mesh = plsc.VectorSubcoreMesh(core_axis_name="c", subcore_axis_name="s")

    n_per_sc = n_per_w * 16
    scratch = (
        [pltpu.VMEM((VV * 128,), jnp.float32),
         pltpu.VMEM_SHARED((2 * n_per_sc,), jnp.int32),
         pltpu.SMEM((2 * chunk,), jnp.int32)]
        + [pltpu.VMEM((chunk * 128,), jnp.float32) for _ in range(_NSLOT)]
        + [pltpu.SemaphoreType.DMA for _ in range(_NSLOT + 1)]
    )

    @functools.partial(
        pl.kernel,
        out_type=jax.ShapeDtypeStruct((N2 * 128,), jnp.float32),
        mesh=mesh,
        scratch_types=scratch,
        compiler_params=pltpu.CompilerParams(needs_layout_passes=False),
    )
    def gather_k(tab_hbm, idx_hbm, out_hbm, *refs):
        tab_v = refs[0]
        idx_sh = refs[1]
        idx_sm = refs[2]
        rows = refs[3:3 + _NSLOT]
        sem_s = refs[3 + _NSLOT:3 + 2 * _NSLOT]
        sem_ld = refs[3 + 2 * _NSLOT]

        cid = lax.axis_index("c")
        sid = lax.axis_index("s")
        wid = cid * 16 + sid
        base = pl.multiple_of(wid * n_per_w, n_per_w)
        sbase = pl.multiple_of(sid * n_per_w, n_per_w)

        pltpu.async_copy(tab_hbm, tab_v, sem_ld)

        @pl.when(sid == 0)
        def _():
            scb = pl.multiple_of(cid * 2 * n_per_sc, 2 * n_per_sc)
            pltpu.sync_copy(idx_hbm.at[pl.ds(scb, 2 * n_per_sc)], idx_sh)

        pltpu.make_async_copy(tab_hbm, tab_v, sem_ld).wait()
        plsc.subcore_barrier()

        def expand_chunk(g, buf):
            pltpu.sync_copy(
                idx_sh.at[pl.ds(2 * (sbase + g * chunk), 2 * chunk)], idx_sm
            )

            @plsc.parallel_loop(0, chunk, 1, unroll=2)
            def pair(p):
                tb = (idx_sm[2 * p] * 16 + idx_sm[2 * p + 1]) * 128
                ob = p * 128
                for k in range(8):
                    buf[pl.ds(ob + k * 16, 16)] = tab_v[pl.ds(tb + k * 16, 16)]

        def fire_store(g, b):
            pltpu.async_copy(
                rows[b],
                out_hbm.at[pl.ds((base + g * chunk) * 128, chunk * 128)],
                sem_s[b],
            )

        def wait_store(b):
            pltpu.make_async_copy(
                rows[b], out_hbm.at[pl.ds(base * 128, chunk * 128)], sem_s[b]
            ).wait()

        def outer(i, carry):
            g0 = i * _NSLOT
            for b in range(_NSLOT):
                g = g0 + b

                @pl.when(i > 0)
                def _():
                    wait_store(b)

                expand_chunk(g, rows[b])
                fire_store(g, b)
            return carry

        lax.fori_loop(0, n_outer, outer, 0)
        for b in range(_NSLOT):
            wait_store(b)

    return gather_k


def kernel(x, emb, W, b, gamma, beta):
    B, L = x.shape
    V, D = emb.shape
    N2 = (B * L) // 2
    paired = _make_paired_table(emb, W, b, gamma, beta).reshape(-1)
    xflat = x.astype(jnp.int32).reshape(-1)
    gather = _make_gather(N2, V * V, n_workers=32, chunk=256)
    out2 = gather(paired, xflat)
    return out2.reshape(B, L, D)
